# baseline (speedup 1.0000x reference)
import dataclasses
import functools

import jax
import jax.numpy as jnp
from jax import lax
from jax.experimental import pallas as pl
from jax.experimental.pallas import tpu as pltpu
from jax.experimental.pallas import tpu_sc as plsc

_D = 1024
_F = 2048
_E = 8
_T = 2048
_TB = 256
_BR = 128
_NB = _T * 2 // _BR + _E
_NP = _NB * _BR
_NW = 32
_TW = _T // _NW
_CH = 16


def _router_body(x_ref, wg_ref, bg_ref, wn_ref, ii_ref, cs_ref, meta_ref,
                 acc_ref):
    i = pl.program_id(0)

    @pl.when(i == 0)
    def _():
        acc_ref[...] = jnp.zeros_like(acc_ref)

    x = x_ref[...]
    logits = jnp.dot(x, wg_ref[...], preferred_element_type=jnp.float32)
    logits = logits + bg_ref[...]
    eio = lax.broadcasted_iota(jnp.int32, (_TB, _E), 1)
    m1 = jnp.max(logits, axis=1, keepdims=True)
    i1 = jnp.min(jnp.where(logits == m1, eio, _E), axis=1)
    oh1 = eio == i1[:, None]
    neg = jnp.where(oh1, -jnp.inf, logits)
    m2 = jnp.max(neg, axis=1, keepdims=True)
    i2 = jnp.min(jnp.where(neg == m2, eio, _E), axis=1)
    oh2 = eio == i2[:, None]
    a = jnp.exp(m2 - m1)[:, 0]
    wn0 = 1.0 / (1.0 + a)
    wn1 = a / (1.0 + a)
    cnt = oh1.astype(jnp.float32) + oh2.astype(jnp.float32)
    rio = lax.broadcasted_iota(jnp.int32, (_TB, _TB), 0)
    cio = lax.broadcasted_iota(jnp.int32, (_TB, _TB), 1)
    tri = (rio > cio).astype(jnp.float32)
    cumb = jnp.dot(tri, cnt, preferred_element_type=jnp.float32)
    cumt = cumb + acc_ref[...]
    cs0 = jnp.sum(jnp.where(oh1, cumt, 0.0), axis=1)
    cs1 = jnp.sum(jnp.where(oh2, cumt, 0.0), axis=1)
    colsum = jnp.sum(cnt, axis=0, keepdims=True)
    ntot = acc_ref[...] + colsum
    acc_ref[...] = ntot
    nb_e = jnp.floor((ntot + (_BR - 1.0)) * (1.0 / _BR))
    kio = lax.broadcasted_iota(jnp.int32, (_E, 16), 0)
    jio = lax.broadcasted_iota(jnp.int32, (_E, 16), 1)
    mtri = (kio < jio).astype(jnp.float32)
    offb16 = jnp.dot(nb_e, mtri, preferred_element_type=jnp.float32)
    meta_ref[...] = jnp.concatenate([offb16, offb16 * _BR], axis=0).astype(jnp.int32)
    wn_ref[...] = jnp.concatenate(
        [wn0.reshape(1, _TB), wn1.reshape(1, _TB)], axis=0)
    ii_ref[...] = jnp.concatenate(
        [i1.reshape(1, _TB), i2.reshape(1, _TB)], axis=0)
    cs_ref[...] = jnp.concatenate(
        [cs0.reshape(1, _TB), cs1.reshape(1, _TB)], axis=0).astype(jnp.int32)


_router_call = pl.pallas_call(
    _router_body,
    grid=(_T // _TB,),
    in_specs=[
        pl.BlockSpec((_TB, _D), lambda i: (i, 0)),
        pl.BlockSpec((_D, _E), lambda i: (0, 0)),
        pl.BlockSpec((1, _E), lambda i: (0, 0)),
    ],
    out_specs=[
        pl.BlockSpec((2, _TB), lambda i: (0, i)),
        pl.BlockSpec((2, _TB), lambda i: (0, i)),
        pl.BlockSpec((2, _TB), lambda i: (0, i)),
        pl.BlockSpec((2, 16), lambda i: (0, 0)),
    ],
    out_shape=[
        jax.ShapeDtypeStruct((2, _T), jnp.float32),
        jax.ShapeDtypeStruct((2, _T), jnp.int32),
        jax.ShapeDtypeStruct((2, _T), jnp.int32),
        jax.ShapeDtypeStruct((2, 16), jnp.int32),
    ],
    scratch_shapes=[pltpu.VMEM((1, _E), jnp.float32)],
)


def _dispatch_body(xf_hbm, idxt_hbm, cst_hbm, meta_hbm, xs_hbm, post_hbm,
                   xbuf, i0v, i1v, c0v, c1v, p0v, p1v, offv, sem):
    wid = lax.axis_index("s") * 2 + lax.axis_index("c")
    base = wid * _TW
    pltpu.sync_copy(meta_hbm.at[1], offv)
    pltpu.sync_copy(idxt_hbm.at[pl.ds(base, _TW)], i0v)
    pltpu.sync_copy(idxt_hbm.at[pl.ds(_T + base, _TW)], i1v)
    pltpu.sync_copy(cst_hbm.at[pl.ds(base, _TW)], c0v)
    pltpu.sync_copy(cst_hbm.at[pl.ds(_T + base, _TW)], c1v)
    for j in range(_TW // 16):
        sl = pl.ds(j * 16, 16)
        p0v[sl] = plsc.load_gather(offv, [i0v[sl]]) + c0v[sl]
        p1v[sl] = plsc.load_gather(offv, [i1v[sl]]) + c1v[sl]
    pltpu.sync_copy(p0v, post_hbm.at[pl.ds(base, _TW)])
    pltpu.sync_copy(p1v, post_hbm.at[pl.ds(_T + base, _TW)])
    pltpu.sync_copy(xf_hbm.at[pl.ds(base, _TW)], xbuf)
    pltpu.async_copy(xbuf, xs_hbm.at[p0v], sem).wait()
    pltpu.async_copy(xbuf, xs_hbm.at[p1v], sem).wait()


def _expert_body(offb_ref, b1_ref, b2_ref, w1_ref, w2_ref, xs_ref, y_ref,
                 w1f, w2f, w1s, w2s, xbuf, ybuf, w1sem, w2sem, xsem, ysem):
    def _w1copy(e, slot):
        return pltpu.make_async_copy(w1_ref.at[e], w1f.at[slot], w1sem.at[slot])

    def _w2copy(e, slot):
        return pltpu.make_async_copy(w2_ref.at[e], w2f.at[slot], w2sem.at[slot])

    _w1copy(0, 0).start()
    _w2copy(0, 0).start()
    for e in range(_E):
        ws = e % 2
        if e + 1 < _E:
            _w1copy(e + 1, 1 - ws).start()
            _w2copy(e + 1, 1 - ws).start()
        _w1copy(e, ws).wait()
        _w2copy(e, ws).wait()
        w1s[...] = w1f[ws].astype(jnp.bfloat16)
        w2s[...] = w2f[ws].astype(jnp.bfloat16)
        b0 = offb_ref[0, e]
        nb = offb_ref[0, e + 1] - b0
        b1e = b1_ref[pl.ds(e, 1), :]
        b2e = b2_ref[pl.ds(e, 1), :]

        def _xcopy(j, slot):
            return pltpu.make_async_copy(
                xs_ref.at[pl.ds((b0 + j) * _BR, _BR)], xbuf.at[slot],
                xsem.at[slot])

        def _ycopy(j, slot):
            return pltpu.make_async_copy(
                ybuf.at[slot], y_ref.at[pl.ds((b0 + j) * _BR, _BR)],
                ysem.at[slot])

        @pl.when(nb > 0)
        def _():
            _xcopy(0, 0).start()

        def body(j, carry):
            slot = jax.lax.rem(j, 2)

            @pl.when(j + 1 < nb)
            def _():
                _xcopy(j + 1, 1 - slot).start()

            _xcopy(j, slot).wait()

            @pl.when(j >= 2)
            def _():
                _ycopy(j - 2, slot).wait()

            x = xbuf[slot].astype(jnp.bfloat16)
            h = jnp.dot(x, w1s[...], preferred_element_type=jnp.float32)
            h = jnp.maximum(h + b1e, 0.0).astype(jnp.bfloat16)
            y = jnp.dot(h, w2s[...], preferred_element_type=jnp.float32)
            ybuf[slot] = y + b2e
            _ycopy(j, slot).start()
            return carry

        jax.lax.fori_loop(0, nb, body, 0)

        @pl.when(nb >= 2)
        def _():
            _ycopy(nb - 2, jax.lax.rem(nb - 2, 2)).wait()

        @pl.when(nb >= 1)
        def _():
            _ycopy(nb - 1, jax.lax.rem(nb - 1, 2)).wait()


_expert_call = pl.pallas_call(
    _expert_body,
    in_specs=[
        pl.BlockSpec(memory_space=pltpu.SMEM),
        pl.BlockSpec(memory_space=pltpu.VMEM),
        pl.BlockSpec(memory_space=pltpu.VMEM),
        pl.BlockSpec(memory_space=pl.ANY),
        pl.BlockSpec(memory_space=pl.ANY),
        pl.BlockSpec(memory_space=pl.ANY),
    ],
    out_specs=pl.BlockSpec(memory_space=pl.ANY),
    scratch_shapes=[
        pltpu.VMEM((2, _D, _F), jnp.float32),
        pltpu.VMEM((2, _F, _D), jnp.float32),
        pltpu.VMEM((_D, _F), jnp.bfloat16),
        pltpu.VMEM((_F, _D), jnp.bfloat16),
        pltpu.VMEM((2, _BR, _D), jnp.float32),
        pltpu.VMEM((2, _BR, _D), jnp.float32),
        pltpu.SemaphoreType.DMA((2,)),
        pltpu.SemaphoreType.DMA((2,)),
        pltpu.SemaphoreType.DMA((2,)),
        pltpu.SemaphoreType.DMA((2,)),
    ],
    out_shape=jax.ShapeDtypeStruct((_NP, _D), jnp.float32),
)


def _combine_body(ys_hbm, post_hbm, wnt_hbm, out_hbm,
                  buf0, buf1, p0v, p1v, w0v, w1v, sem0, sem1, osem):
    wid = lax.axis_index("s") * 2 + lax.axis_index("c")
    base = wid * _TW
    nch = _TW // _CH
    pltpu.sync_copy(wnt_hbm.at[pl.ds(base, _TW)], w0v)
    pltpu.sync_copy(wnt_hbm.at[pl.ds(_T + base, _TW)], w1v)

    def _load_pos(c, slot):
        tb = base + c * _CH
        pltpu.sync_copy(post_hbm.at[pl.ds(tb, _CH)], p0v.at[slot])
        pltpu.sync_copy(post_hbm.at[pl.ds(_T + tb, _CH)], p1v.at[slot])

    def _start_gather(slot):
        cp0 = pltpu.async_copy(ys_hbm.at[p0v.at[slot]], buf0.at[slot], sem0.at[slot])
        cp1 = pltpu.async_copy(ys_hbm.at[p1v.at[slot]], buf1.at[slot], sem1.at[slot])
        return cp0, cp1

    def _gather_wait(slot):
        pltpu.make_async_copy(ys_hbm.at[p0v.at[slot]], buf0.at[slot],
                              sem0.at[slot]).wait()
        pltpu.make_async_copy(ys_hbm.at[p1v.at[slot]], buf1.at[slot],
                              sem1.at[slot]).wait()

    def _ocopy(c, slot):
        return pltpu.make_async_copy(
            buf0.at[slot], out_hbm.at[pl.ds(base + c * _CH, _CH)], osem.at[slot])

    _load_pos(0, 0)
    _start_gather(0)
    for c in range(nch):
        slot = c % 2
        if c + 1 < nch:
            _load_pos(c + 1, 1 - slot)
            _start_gather(1 - slot)
        _gather_wait(slot)
        if c >= 2:
            _ocopy(c - 2, slot).wait()

        @pl.loop(0, _CH)
        def _(i):
            tok = jnp.full((16,), c * _CH, jnp.int32) + i
            w0 = plsc.load_gather(w0v, [tok])
            w1 = plsc.load_gather(w1v, [tok])
            for j in range(_D // 16):
                sl = pl.ds(j * 16, 16)
                buf0[slot, i, sl] = buf0[slot, i, sl] * w0 + buf1[slot, i, sl] * w1

        _ocopy(c, slot).start()
    _ocopy(nch - 2, nch % 2).wait()
    _ocopy(nch - 1, (nch - 1) % 2).wait()


@functools.lru_cache(maxsize=1)
def _sc_calls():
    mesh = plsc.VectorSubcoreMesh(core_axis_name="c", subcore_axis_name="s")
    cp = pltpu.CompilerParams()
    if "needs_layout_passes" in pltpu.CompilerParams.__dataclass_fields__:
        cp = dataclasses.replace(cp, needs_layout_passes=False)
    dispatch = pl.kernel(
        _dispatch_body,
        out_type=(
            jax.ShapeDtypeStruct((_NP, _D), jnp.float32),
            jax.ShapeDtypeStruct((2 * _T,), jnp.int32),
        ),
        mesh=mesh,
        scratch_types=[
            pltpu.VMEM((_TW, _D), jnp.float32),
            pltpu.VMEM((_TW,), jnp.int32),
            pltpu.VMEM((_TW,), jnp.int32),
            pltpu.VMEM((_TW,), jnp.int32),
            pltpu.VMEM((_TW,), jnp.int32),
            pltpu.VMEM((_TW,), jnp.int32),
            pltpu.VMEM((_TW,), jnp.int32),
            pltpu.VMEM((16,), jnp.int32),
            pltpu.SemaphoreType.DMA,
        ],
        compiler_params=cp,
    )
    combine = pl.kernel(
        _combine_body,
        out_type=jax.ShapeDtypeStruct((_T, _D), jnp.float32),
        mesh=mesh,
        scratch_types=[
            pltpu.VMEM((2, _CH, _D), jnp.float32),
            pltpu.VMEM((2, _CH, _D), jnp.float32),
            pltpu.VMEM((2, _CH), jnp.int32),
            pltpu.VMEM((2, _CH), jnp.int32),
            pltpu.VMEM((_TW,), jnp.float32),
            pltpu.VMEM((_TW,), jnp.float32),
            pltpu.SemaphoreType.DMA((2,)),
            pltpu.SemaphoreType.DMA((2,)),
            pltpu.SemaphoreType.DMA((2,)),
        ],
        compiler_params=cp,
    )
    return dispatch, combine


def _wsum_body(w1_ref, w2_ref, o_ref):
    i = pl.program_id(0)

    @pl.when(i == 0)
    def _():
        o_ref[...] = jnp.zeros_like(o_ref)

    o_ref[...] += (jnp.sum(w1_ref[0], axis=0, keepdims=True)[:, :8]
                   + jnp.sum(w2_ref[0], axis=0, keepdims=True)[:, :8])


_wsum_call = pl.pallas_call(
    _wsum_body,
    grid=(_E,),
    in_specs=[
        pl.BlockSpec((1, _D, _F), lambda i: (i, 0, 0)),
        pl.BlockSpec((1, _F, _D), lambda i: (i, 0, 0)),
    ],
    out_specs=pl.BlockSpec((1, 8), lambda i: (0, 0)),
    out_shape=jax.ShapeDtypeStruct((1, 8), jnp.float32),
)


def kernel(x, W1, b1, W2, b2, Wg, bg):
    B, S, D = x.shape
    if _PROBE == 3:
        return _wsum_call(W1, W2)
    xf = x.reshape(_T, _D)
    wn2, ii2, cs2, meta = _router_call(xf, Wg, bg.reshape(1, _E))
    wnt = wn2.reshape(2 * _T)
    idxt = ii2.reshape(2 * _T)
    cst = cs2.reshape(2 * _T)
    dispatch, combine = _sc_calls()
    xs, post = dispatch(xf, idxt, cst, meta)
    if _PROBE == 1:
        return xs
    ys = _expert_call(meta, b1, b2, W1, W2, xs)
    if _PROBE == 2:
        return ys
    out = combine(ys, post, wnt)
    return out.reshape(B, S, D)


_PROBE = 0


# speedup vs baseline: 1.0015x; 1.0015x over previous
import dataclasses
import functools

import jax
import jax.numpy as jnp
from jax import lax
from jax.experimental import pallas as pl
from jax.experimental.pallas import tpu as pltpu
from jax.experimental.pallas import tpu_sc as plsc

_D = 1024
_F = 2048
_E = 8
_T = 2048
_TB = 256
_BR = 128
_NB = _T * 2 // _BR + _E
_NP = _NB * _BR
_NW = 32
_TW = _T // _NW
_CH = 16


def _router_body(x_ref, wg_ref, bg_ref, wn_ref, ii_ref, cs_ref, meta_ref,
                 acc_ref):
    i = pl.program_id(0)

    @pl.when(i == 0)
    def _():
        acc_ref[...] = jnp.zeros_like(acc_ref)

    x = x_ref[...]
    logits = jnp.dot(x, wg_ref[...], preferred_element_type=jnp.float32)
    logits = logits + bg_ref[...]
    eio = lax.broadcasted_iota(jnp.int32, (_TB, _E), 1)
    m1 = jnp.max(logits, axis=1, keepdims=True)
    i1 = jnp.min(jnp.where(logits == m1, eio, _E), axis=1)
    oh1 = eio == i1[:, None]
    neg = jnp.where(oh1, -jnp.inf, logits)
    m2 = jnp.max(neg, axis=1, keepdims=True)
    i2 = jnp.min(jnp.where(neg == m2, eio, _E), axis=1)
    oh2 = eio == i2[:, None]
    a = jnp.exp(m2 - m1)[:, 0]
    wn0 = 1.0 / (1.0 + a)
    wn1 = a / (1.0 + a)
    cnt = oh1.astype(jnp.float32) + oh2.astype(jnp.float32)
    rio = lax.broadcasted_iota(jnp.int32, (_TB, _TB), 0)
    cio = lax.broadcasted_iota(jnp.int32, (_TB, _TB), 1)
    tri = (rio > cio).astype(jnp.float32)
    cumb = jnp.dot(tri, cnt, preferred_element_type=jnp.float32)
    cumt = cumb + acc_ref[...]
    cs0 = jnp.sum(jnp.where(oh1, cumt, 0.0), axis=1)
    cs1 = jnp.sum(jnp.where(oh2, cumt, 0.0), axis=1)
    colsum = jnp.sum(cnt, axis=0, keepdims=True)
    ntot = acc_ref[...] + colsum
    acc_ref[...] = ntot
    nb_e = jnp.floor((ntot + (_BR - 1.0)) * (1.0 / _BR))
    kio = lax.broadcasted_iota(jnp.int32, (_E, 16), 0)
    jio = lax.broadcasted_iota(jnp.int32, (_E, 16), 1)
    mtri = (kio < jio).astype(jnp.float32)
    offb16 = jnp.dot(nb_e, mtri, preferred_element_type=jnp.float32)
    meta_ref[...] = jnp.concatenate([offb16, offb16 * _BR], axis=0).astype(jnp.int32)
    wn_ref[...] = jnp.concatenate(
        [wn0.reshape(1, _TB), wn1.reshape(1, _TB)], axis=0)
    ii_ref[...] = jnp.concatenate(
        [i1.reshape(1, _TB), i2.reshape(1, _TB)], axis=0)
    cs_ref[...] = jnp.concatenate(
        [cs0.reshape(1, _TB), cs1.reshape(1, _TB)], axis=0).astype(jnp.int32)


_router_call = pl.pallas_call(
    _router_body,
    grid=(_T // _TB,),
    in_specs=[
        pl.BlockSpec((_TB, _D), lambda i: (i, 0)),
        pl.BlockSpec((_D, _E), lambda i: (0, 0)),
        pl.BlockSpec((1, _E), lambda i: (0, 0)),
    ],
    out_specs=[
        pl.BlockSpec((2, _TB), lambda i: (0, i)),
        pl.BlockSpec((2, _TB), lambda i: (0, i)),
        pl.BlockSpec((2, _TB), lambda i: (0, i)),
        pl.BlockSpec((2, 16), lambda i: (0, 0)),
    ],
    out_shape=[
        jax.ShapeDtypeStruct((2, _T), jnp.float32),
        jax.ShapeDtypeStruct((2, _T), jnp.int32),
        jax.ShapeDtypeStruct((2, _T), jnp.int32),
        jax.ShapeDtypeStruct((2, 16), jnp.int32),
    ],
    scratch_shapes=[pltpu.VMEM((1, _E), jnp.float32)],
)


def _dispatch_body(xf_hbm, idxt_hbm, cst_hbm, meta_hbm, xs_hbm, post_hbm,
                   xbuf, i0v, i1v, c0v, c1v, p0v, p1v, offv, sem):
    wid = lax.axis_index("s") * 2 + lax.axis_index("c")
    base = wid * _TW
    pltpu.sync_copy(meta_hbm.at[1], offv)
    pltpu.sync_copy(idxt_hbm.at[pl.ds(base, _TW)], i0v)
    pltpu.sync_copy(idxt_hbm.at[pl.ds(_T + base, _TW)], i1v)
    pltpu.sync_copy(cst_hbm.at[pl.ds(base, _TW)], c0v)
    pltpu.sync_copy(cst_hbm.at[pl.ds(_T + base, _TW)], c1v)
    for j in range(_TW // 16):
        sl = pl.ds(j * 16, 16)
        p0v[sl] = plsc.load_gather(offv, [i0v[sl]]) + c0v[sl]
        p1v[sl] = plsc.load_gather(offv, [i1v[sl]]) + c1v[sl]
    pltpu.sync_copy(p0v, post_hbm.at[pl.ds(base, _TW)])
    pltpu.sync_copy(p1v, post_hbm.at[pl.ds(_T + base, _TW)])
    pltpu.sync_copy(xf_hbm.at[pl.ds(base, _TW)], xbuf)
    pltpu.async_copy(xbuf, xs_hbm.at[p0v], sem).wait()
    pltpu.async_copy(xbuf, xs_hbm.at[p1v], sem).wait()


def _expert_body(offb_ref, b1_ref, b2_ref, w1_ref, w2_ref, xs_ref, y_ref,
                 w1f, w2f, w1s, w2s, xbuf, ybuf, w1sem, w2sem, xsem, ysem):
    def _w1copy(e, slot):
        return pltpu.make_async_copy(w1_ref.at[e], w1f.at[slot], w1sem.at[slot])

    def _w2copy(e, slot):
        return pltpu.make_async_copy(w2_ref.at[e], w2f.at[slot], w2sem.at[slot])

    _w1copy(0, 0).start()
    _w2copy(0, 0).start()
    for e in range(_E):
        ws = e % 2
        if e + 1 < _E:
            _w1copy(e + 1, 1 - ws).start()
            _w2copy(e + 1, 1 - ws).start()
        _w1copy(e, ws).wait()
        _w2copy(e, ws).wait()
        w1s[...] = w1f[ws].astype(jnp.bfloat16)
        w2s[...] = w2f[ws].astype(jnp.bfloat16)
        b0 = offb_ref[0, e]
        nb = offb_ref[0, e + 1] - b0
        b1e = b1_ref[pl.ds(e, 1), :]
        b2e = b2_ref[pl.ds(e, 1), :]

        def _xcopy(j, slot):
            return pltpu.make_async_copy(
                xs_ref.at[pl.ds((b0 + j) * _BR, _BR)], xbuf.at[slot],
                xsem.at[slot])

        def _ycopy(j, slot):
            return pltpu.make_async_copy(
                ybuf.at[slot], y_ref.at[pl.ds((b0 + j) * _BR, _BR)],
                ysem.at[slot])

        @pl.when(nb > 0)
        def _():
            _xcopy(0, 0).start()

        def body(j, carry):
            slot = jax.lax.rem(j, 2)

            @pl.when(j + 1 < nb)
            def _():
                _xcopy(j + 1, 1 - slot).start()

            _xcopy(j, slot).wait()

            @pl.when(j >= 2)
            def _():
                _ycopy(j - 2, slot).wait()

            x = xbuf[slot].astype(jnp.bfloat16)
            h = jnp.dot(x, w1s[...], preferred_element_type=jnp.float32)
            h = jnp.maximum(h + b1e, 0.0).astype(jnp.bfloat16)
            y = jnp.dot(h, w2s[...], preferred_element_type=jnp.float32)
            ybuf[slot] = y + b2e
            _ycopy(j, slot).start()
            return carry

        jax.lax.fori_loop(0, nb, body, 0)

        @pl.when(nb >= 2)
        def _():
            _ycopy(nb - 2, jax.lax.rem(nb - 2, 2)).wait()

        @pl.when(nb >= 1)
        def _():
            _ycopy(nb - 1, jax.lax.rem(nb - 1, 2)).wait()


_expert_call = pl.pallas_call(
    _expert_body,
    in_specs=[
        pl.BlockSpec(memory_space=pltpu.SMEM),
        pl.BlockSpec(memory_space=pltpu.VMEM),
        pl.BlockSpec(memory_space=pltpu.VMEM),
        pl.BlockSpec(memory_space=pl.ANY),
        pl.BlockSpec(memory_space=pl.ANY),
        pl.BlockSpec(memory_space=pl.ANY),
    ],
    out_specs=pl.BlockSpec(memory_space=pl.ANY),
    scratch_shapes=[
        pltpu.VMEM((2, _D, _F), jnp.float32),
        pltpu.VMEM((2, _F, _D), jnp.float32),
        pltpu.VMEM((_D, _F), jnp.bfloat16),
        pltpu.VMEM((_F, _D), jnp.bfloat16),
        pltpu.VMEM((2, _BR, _D), jnp.float32),
        pltpu.VMEM((2, _BR, _D), jnp.float32),
        pltpu.SemaphoreType.DMA((2,)),
        pltpu.SemaphoreType.DMA((2,)),
        pltpu.SemaphoreType.DMA((2,)),
        pltpu.SemaphoreType.DMA((2,)),
    ],
    out_shape=jax.ShapeDtypeStruct((_NP, _D), jnp.float32),
)


def _combine_body(ys_hbm, post_hbm, wnt_hbm, out_hbm,
                  buf0, buf1, p0v, p1v, w0v, w1v, sem0, sem1, osem):
    wid = lax.axis_index("s") * 2 + lax.axis_index("c")
    base = wid * _TW
    nch = _TW // _CH
    pltpu.sync_copy(wnt_hbm.at[pl.ds(base, _TW)], w0v)
    pltpu.sync_copy(wnt_hbm.at[pl.ds(_T + base, _TW)], w1v)

    def _load_pos(c, slot):
        tb = base + c * _CH
        pltpu.sync_copy(post_hbm.at[pl.ds(tb, _CH)], p0v.at[slot])
        pltpu.sync_copy(post_hbm.at[pl.ds(_T + tb, _CH)], p1v.at[slot])

    def _start_gather(slot):
        cp0 = pltpu.async_copy(ys_hbm.at[p0v.at[slot]], buf0.at[slot], sem0.at[slot])
        cp1 = pltpu.async_copy(ys_hbm.at[p1v.at[slot]], buf1.at[slot], sem1.at[slot])
        return cp0, cp1

    def _gather_wait(slot):
        pltpu.make_async_copy(ys_hbm.at[p0v.at[slot]], buf0.at[slot],
                              sem0.at[slot]).wait()
        pltpu.make_async_copy(ys_hbm.at[p1v.at[slot]], buf1.at[slot],
                              sem1.at[slot]).wait()

    def _ocopy(c, slot):
        return pltpu.make_async_copy(
            buf0.at[slot], out_hbm.at[pl.ds(base + c * _CH, _CH)], osem.at[slot])

    _load_pos(0, 0)
    _start_gather(0)
    for c in range(nch):
        slot = c % 2
        if c + 1 < nch:
            _load_pos(c + 1, 1 - slot)
            _start_gather(1 - slot)
        _gather_wait(slot)
        if c >= 2:
            _ocopy(c - 2, slot).wait()

        @pl.loop(0, _CH)
        def _(i):
            tok = jnp.full((16,), c * _CH, jnp.int32) + i
            w0 = plsc.load_gather(w0v, [tok])
            w1 = plsc.load_gather(w1v, [tok])
            for j in range(_D // 16):
                sl = pl.ds(j * 16, 16)
                buf0[slot, i, sl] = buf0[slot, i, sl] * w0 + buf1[slot, i, sl] * w1

        _ocopy(c, slot).start()
    _ocopy(nch - 2, nch % 2).wait()
    _ocopy(nch - 1, (nch - 1) % 2).wait()


@functools.lru_cache(maxsize=1)
def _sc_calls():
    mesh = plsc.VectorSubcoreMesh(core_axis_name="c", subcore_axis_name="s")
    cp = pltpu.CompilerParams()
    if "needs_layout_passes" in pltpu.CompilerParams.__dataclass_fields__:
        cp = dataclasses.replace(cp, needs_layout_passes=False)
    dispatch = pl.kernel(
        _dispatch_body,
        out_type=(
            jax.ShapeDtypeStruct((_NP, _D), jnp.float32),
            jax.ShapeDtypeStruct((2 * _T,), jnp.int32),
        ),
        mesh=mesh,
        scratch_types=[
            pltpu.VMEM((_TW, _D), jnp.float32),
            pltpu.VMEM((_TW,), jnp.int32),
            pltpu.VMEM((_TW,), jnp.int32),
            pltpu.VMEM((_TW,), jnp.int32),
            pltpu.VMEM((_TW,), jnp.int32),
            pltpu.VMEM((_TW,), jnp.int32),
            pltpu.VMEM((_TW,), jnp.int32),
            pltpu.VMEM((16,), jnp.int32),
            pltpu.SemaphoreType.DMA,
        ],
        compiler_params=cp,
    )
    combine = pl.kernel(
        _combine_body,
        out_type=jax.ShapeDtypeStruct((_T, _D), jnp.float32),
        mesh=mesh,
        scratch_types=[
            pltpu.VMEM((2, _CH, _D), jnp.float32),
            pltpu.VMEM((2, _CH, _D), jnp.float32),
            pltpu.VMEM((2, _CH), jnp.int32),
            pltpu.VMEM((2, _CH), jnp.int32),
            pltpu.VMEM((_TW,), jnp.float32),
            pltpu.VMEM((_TW,), jnp.float32),
            pltpu.SemaphoreType.DMA((2,)),
            pltpu.SemaphoreType.DMA((2,)),
            pltpu.SemaphoreType.DMA((2,)),
        ],
        compiler_params=cp,
    )
    return dispatch, combine


def kernel(x, W1, b1, W2, b2, Wg, bg):
    B, S, D = x.shape
    xf = x.reshape(_T, _D)
    wn2, ii2, cs2, meta = _router_call(xf, Wg, bg.reshape(1, _E))
    wnt = wn2.reshape(2 * _T)
    idxt = ii2.reshape(2 * _T)
    cst = cs2.reshape(2 * _T)
    dispatch, combine = _sc_calls()
    xs, post = dispatch(xf, idxt, cst, meta)
    ys = _expert_call(meta, b1, b2, W1, W2, xs)
    out = combine(ys, post, wnt)
    return out.reshape(B, S, D)
